# baseline (speedup 1.0000x reference)
import functools
import math

import jax
import jax.numpy as jnp
import numpy as np
from jax import lax
from jax.experimental import pallas as pl
from jax.experimental.pallas import tpu as pltpu, tpu_sc as plsc

L = 2048
D_MODEL = 768
N_HEADS = 12
D_HEAD = 64
N_SEL = 512
R = 256
NEG = -1e30
NROW = L * N_HEADS
HALF = N_SEL // 2
NSP = N_SEL + 8
CH = NROW // 32


def _build_counts_t() -> np.ndarray:
    try:
        cpu = jax.devices("cpu")[0]
        ctx = jax.default_device(cpu)
    except Exception:
        import contextlib
        ctx = contextlib.nullcontext()
    with ctx:
        idx = np.asarray(
            jax.random.randint(jax.random.key(42), (L, N_SEL), 0, L))
    counts = np.zeros((L, L), np.float32)
    np.add.at(counts, (np.arange(L)[:, None], idx), 1.0)
    return np.ascontiguousarray(counts.T)


_COUNTS_T = _build_counts_t()


def _proj_body(xq_ref, xk_ref, xv_ref, wq_ref, bq_ref, wk_ref, bk_ref,
               wv_ref, bv_ref, q_ref, k_ref, v_ref):
    q_ref[...] = (
        jnp.dot(xq_ref[...], wq_ref[...], preferred_element_type=jnp.float32)
        + bq_ref[...])
    k_ref[...] = (
        jnp.dot(xk_ref[...], wk_ref[...], preferred_element_type=jnp.float32)
        + bk_ref[...])
    v_ref[...] = (
        jnp.dot(xv_ref[...], wv_ref[...], preferred_element_type=jnp.float32)
        + bv_ref[...])


def _measure_body(q_ref, k_ref, ct_ref, m_ref):
    qt = pl.program_id(0)
    c = ct_ref[...]
    sampled = c > 0.0
    for h in range(N_HEADS):
        kh = k_ref[:, h * D_HEAD:(h + 1) * D_HEAD]
        qh = q_ref[:, h * D_HEAD:(h + 1) * D_HEAD]
        s_t = jax.lax.dot_general(
            kh, qh, (((1,), (1,)), ((), ())),
            preferred_element_type=jnp.float32)
        mx = jnp.max(jnp.where(sampled, s_t, NEG), axis=0)
        sm = jnp.sum(s_t * c, axis=0)
        m_ref[h, pl.ds(qt * R, R)] = mx - sm * (1.0 / L)


def _scan_rows(x):
    incl = x
    sh = 1
    while sh < L:
        incl = incl + jnp.concatenate(
            [jnp.zeros((N_HEADS, sh), jnp.float32), incl[:, :L - sh]], axis=1)
        sh *= 2
    return incl


def _select_body(m_ref, sel_ref, pos_ref):
    m = m_ref[...]
    lo = jnp.min(m, axis=1, keepdims=True) - 1.0
    hi = jnp.max(m, axis=1, keepdims=True)
    kf = float(N_SEL)

    def step(_, carry):
        lo, hi = carry
        mid = 0.5 * (lo + hi)
        cnt = jnp.sum((m > mid).astype(jnp.float32), axis=1, keepdims=True)
        big = cnt >= kf
        return jnp.where(big, mid, lo), jnp.where(big, hi, mid)

    lo, hi = jax.lax.fori_loop(0, 60, step, (lo, hi))
    thr = jnp.max(jnp.where(m <= hi, m, NEG), axis=1, keepdims=True)
    gt = (m > thr).astype(jnp.float32)
    need = kf - jnp.sum(gt, axis=1, keepdims=True)
    tie = (m == thr).astype(jnp.float32)
    tie_excl = _scan_rows(tie) - tie
    sel = gt + tie * (tie_excl < need).astype(jnp.float32)
    sel_ref[...] = sel
    pos_ref[...] = _scan_rows(sel) - sel


_SC_MESH = plsc.VectorSubcoreMesh(core_axis_name="c", subcore_axis_name="s")


@functools.partial(
    pl.kernel, mesh=_SC_MESH,
    compiler_params=pltpu.CompilerParams(needs_layout_passes=False),
    out_type=[
        jax.ShapeDtypeStruct((N_HEADS, N_SEL, 2 * D_HEAD), jnp.float32),
        jax.ShapeDtypeStruct((N_HEADS, N_SEL), jnp.int32),
    ],
    scratch_types=[
        pltpu.VMEM((L,), jnp.float32),
        pltpu.VMEM((L,), jnp.float32),
        pltpu.VMEM((HALF,), jnp.float32),
        pltpu.VMEM((HALF,), jnp.int32),
        pltpu.VMEM((HALF,), jnp.int32),
        pltpu.VMEM((HALF, 2 * D_HEAD), jnp.float32),
        pltpu.SemaphoreType.DMA,
    ],
)
def _sc_gather(sel_hbm, pos_hbm, q2_hbm, qsel_hbm, idx_hbm,
               sel_v, pos_v, idxf_v, idx_v, gidx_v, rows_v, sem):
    wid = lax.axis_index("s") * 2 + lax.axis_index("c")
    h = wid // 2
    hf = wid % 2

    @pl.when(h < N_HEADS)
    def _():
        pltpu.sync_copy(sel_hbm.at[h], sel_v)
        pltpu.sync_copy(pos_hbm.at[h], pos_v)
        base = jnp.float32(hf * HALF)

        def body(i, _):
            s16 = sel_v[pl.ds(i * 16, 16)]
            p16 = pos_v[pl.ds(i * 16, 16)]
            i16 = lax.iota(jnp.int32, 16) + i * 16
            m = (s16 > 0.5) & (p16 >= base) & (p16 < base + HALF)
            dst = p16.astype(jnp.int32) - hf * HALF
            plsc.store_scatter(idxf_v, [dst],
                               (i16 * N_HEADS + h).astype(jnp.float32),
                               mask=m)
            return 0

        lax.fori_loop(0, L // 16, body, 0)

        def conv(i, _):
            v = idxf_v[pl.ds(i * 16, 16)].astype(jnp.int32)
            idx_v[pl.ds(i * 16, 16)] = v
            gidx_v[pl.ds(i * 16, 16)] = v // 2
            return 0

        lax.fori_loop(0, HALF // 16, conv, 0)
        for j in range(HALF // 128):
            pltpu.async_copy(
                q2_hbm.at[gidx_v.at[pl.ds(j * 128, 128)]],
                rows_v.at[pl.ds(j * 128, 128)], sem).wait()
        pltpu.sync_copy(rows_v, qsel_hbm.at[h, pl.ds(hf * HALF, HALF)])
        pltpu.sync_copy(idx_v, idx_hbm.at[h, pl.ds(hf * HALF, HALF)])


def _attn_body(qs_ref, k_ref, v_ref, out_ref):
    scale = 1.0 / math.sqrt(D_HEAD)
    for hh in range(2):
        sl = slice(hh * D_HEAD, (hh + 1) * D_HEAD)
        qs = qs_ref[hh][:, sl]
        s = jax.lax.dot_general(
            qs, k_ref[:, sl], (((1,), (1,)), ((), ())),
            preferred_element_type=jnp.float32) * scale
        mx = jnp.max(s, axis=1, keepdims=True)
        e = jnp.exp(s - mx)
        den = jnp.sum(e, axis=1, keepdims=True)
        attn = jnp.dot(e, v_ref[:, sl],
                       preferred_element_type=jnp.float32) / den
        vmean = jnp.mean(v_ref[:, sl], axis=0, keepdims=True)
        vpad = jnp.broadcast_to(vmean, (NSP - N_SEL, D_HEAD))
        out_ref[hh, 0:N_SEL, 0:D_HEAD] = attn
        out_ref[hh, 0:N_SEL, D_HEAD:2 * D_HEAD] = attn
        out_ref[hh, N_SEL:NSP, 0:D_HEAD] = vpad
        out_ref[hh, N_SEL:NSP, D_HEAD:2 * D_HEAD] = vpad


@functools.partial(
    pl.kernel, mesh=_SC_MESH,
    compiler_params=pltpu.CompilerParams(needs_layout_passes=False),
    out_type=jax.ShapeDtypeStruct((NROW, 2 * D_HEAD), jnp.float32),
    scratch_types=[
        pltpu.VMEM((N_HEADS * N_SEL,), jnp.int32),
        pltpu.VMEM((CH,), jnp.float32),
        pltpu.VMEM((CH,), jnp.int32),
        pltpu.VMEM((CH, 2 * D_HEAD), jnp.float32),
        pltpu.SemaphoreType.DMA,
    ],
)
def _sc_assemble(idxf_hbm, attnf_hbm, out_hbm, idxall_v, mapf_v, map_v,
                 chunk_v, sem):
    wid = lax.axis_index("s") * 2 + lax.axis_index("c")
    lo = wid * CH
    pltpu.sync_copy(idxf_hbm, idxall_v)

    def dbody(i, _):
        r16 = lax.iota(jnp.int32, 16) + (lo + i * 16)
        hmod = lax.rem(r16, N_HEADS)
        mapf_v[pl.ds(i * 16, 16)] = (hmod * NSP + N_SEL).astype(jnp.float32)
        return 0

    lax.fori_loop(0, CH // 16, dbody, 0)

    for h in range(N_HEADS):
        def sbody(j, _):
            idx16 = idxall_v[pl.ds(h * N_SEL + j * 16, 16)]
            val16 = lax.iota(jnp.int32, 16) + (h * NSP + j * 16)
            m = (idx16 >= lo) & (idx16 < lo + CH)
            plsc.store_scatter(mapf_v, [idx16 - lo],
                               val16.astype(jnp.float32), mask=m)
            return 0

        lax.fori_loop(0, N_SEL // 16, sbody, 0)

    def conv(i, _):
        map_v[pl.ds(i * 16, 16)] = mapf_v[pl.ds(i * 16, 16)].astype(jnp.int32)
        return 0

    lax.fori_loop(0, CH // 16, conv, 0)

    copies = [
        pltpu.async_copy(
            attnf_hbm.at[map_v.at[pl.ds(j * 128, 128)]],
            chunk_v.at[pl.ds(j * 128, 128)], sem)
        for j in range(CH // 128)
    ]
    for c in copies:
        c.wait()
    pltpu.sync_copy(chunk_v, out_hbm.at[pl.ds(lo, CH)])


def _compact_body(w_ref, out_ref):
    for h in range(N_HEADS):
        out_ref[:, h * D_HEAD:(h + 1) * D_HEAD] = w_ref[:, h, 0:D_HEAD]


def kernel(query, key, value, Wq, bq, Wk, bk, Wv, bv):
    xq = query[0]
    xk = key[0]
    xv = value[0]
    b2 = lambda b: b.reshape(1, D_MODEL)
    counts_t = jnp.asarray(_COUNTS_T)

    q, k, v = pl.pallas_call(
        _proj_body,
        grid=(L // R,),
        in_specs=[
            pl.BlockSpec((R, D_MODEL), lambda i: (i, 0)),
            pl.BlockSpec((R, D_MODEL), lambda i: (i, 0)),
            pl.BlockSpec((R, D_MODEL), lambda i: (i, 0)),
            pl.BlockSpec((D_MODEL, D_MODEL), lambda i: (0, 0)),
            pl.BlockSpec((1, D_MODEL), lambda i: (0, 0)),
            pl.BlockSpec((D_MODEL, D_MODEL), lambda i: (0, 0)),
            pl.BlockSpec((1, D_MODEL), lambda i: (0, 0)),
            pl.BlockSpec((D_MODEL, D_MODEL), lambda i: (0, 0)),
            pl.BlockSpec((1, D_MODEL), lambda i: (0, 0)),
        ],
        out_specs=[
            pl.BlockSpec((R, D_MODEL), lambda i: (i, 0)),
            pl.BlockSpec((R, D_MODEL), lambda i: (i, 0)),
            pl.BlockSpec((R, D_MODEL), lambda i: (i, 0)),
        ],
        out_shape=[jax.ShapeDtypeStruct((L, D_MODEL), jnp.float32)] * 3,
    )(xq, xk, xv, Wq, b2(bq), Wk, b2(bk), Wv, b2(bv))

    measure = pl.pallas_call(
        _measure_body,
        grid=(L // R,),
        in_specs=[
            pl.BlockSpec((R, D_MODEL), lambda i: (i, 0)),
            pl.BlockSpec((L, D_MODEL), lambda i: (0, 0)),
            pl.BlockSpec((L, R), lambda i: (0, i)),
        ],
        out_specs=pl.BlockSpec((N_HEADS, L), lambda i: (0, 0)),
        out_shape=jax.ShapeDtypeStruct((N_HEADS, L), jnp.float32),
    )(q, k, counts_t)

    sel, pos = pl.pallas_call(
        _select_body,
        out_shape=[jax.ShapeDtypeStruct((N_HEADS, L), jnp.float32)] * 2,
    )(measure)

    q2 = q.reshape(L * N_HEADS // 2, 2 * D_HEAD)
    q_sel, idx = _sc_gather(sel, pos, q2)

    attn_ext = pl.pallas_call(
        _attn_body,
        grid=(N_HEADS // 2,),
        in_specs=[
            pl.BlockSpec((2, N_SEL, 2 * D_HEAD), lambda h: (h, 0, 0)),
            pl.BlockSpec((L, 2 * D_HEAD), lambda h: (0, h)),
            pl.BlockSpec((L, 2 * D_HEAD), lambda h: (0, h)),
        ],
        out_specs=pl.BlockSpec((2, NSP, 2 * D_HEAD), lambda h: (h, 0, 0)),
        out_shape=jax.ShapeDtypeStruct((N_HEADS, NSP, 2 * D_HEAD),
                                       jnp.float32),
    )(q_sel, k, v)

    outw = _sc_assemble(idx.reshape(-1),
                        attn_ext.reshape(N_HEADS * NSP, 2 * D_HEAD))

    out = pl.pallas_call(
        _compact_body,
        grid=(L // R,),
        in_specs=[
            pl.BlockSpec((R, N_HEADS, 2 * D_HEAD), lambda i: (i, 0, 0)),
        ],
        out_specs=pl.BlockSpec((R, D_MODEL), lambda i: (i, 0)),
        out_shape=jax.ShapeDtypeStruct((L, D_MODEL), jnp.float32),
    )(outw.reshape(L, N_HEADS, 2 * D_HEAD))

    return out[None]


# speedup vs baseline: 1.0024x; 1.0024x over previous
import functools
import math

import jax
import jax.numpy as jnp
import numpy as np
from jax import lax
from jax.experimental import pallas as pl
from jax.experimental.pallas import tpu as pltpu, tpu_sc as plsc

L = 2048
D_MODEL = 768
N_HEADS = 12
D_HEAD = 64
N_SEL = 512
R = 256
NEG = -1e30
NROW = L * N_HEADS
HALF = N_SEL // 2
NSP = N_SEL + 8
CH = NROW // 32


def _build_counts_t() -> np.ndarray:
    try:
        cpu = jax.devices("cpu")[0]
        ctx = jax.default_device(cpu)
    except Exception:
        import contextlib
        ctx = contextlib.nullcontext()
    with ctx:
        idx = np.asarray(
            jax.random.randint(jax.random.key(42), (L, N_SEL), 0, L))
    counts = np.zeros((L, L), np.float32)
    np.add.at(counts, (np.arange(L)[:, None], idx), 1.0)
    return np.ascontiguousarray(counts.T)


_COUNTS_T = _build_counts_t()
_MAP_DEFAULT = (((np.arange(L * N_HEADS) % N_HEADS) * (N_SEL + 8) + N_SEL)
                .astype(np.float32))


def _proj_body(xq_ref, xk_ref, xv_ref, wq_ref, bq_ref, wk_ref, bk_ref,
               wv_ref, bv_ref, q_ref, k_ref, v_ref):
    q_ref[...] = (
        jnp.dot(xq_ref[...], wq_ref[...], preferred_element_type=jnp.float32)
        + bq_ref[...])
    k_ref[...] = (
        jnp.dot(xk_ref[...], wk_ref[...], preferred_element_type=jnp.float32)
        + bk_ref[...])
    v_ref[...] = (
        jnp.dot(xv_ref[...], wv_ref[...], preferred_element_type=jnp.float32)
        + bv_ref[...])


def _measure_body(q_ref, k_ref, ct_ref, m_ref):
    qt = pl.program_id(0)
    c = ct_ref[...]
    sampled = c > 0.0
    for h in range(N_HEADS):
        kh = k_ref[:, h * D_HEAD:(h + 1) * D_HEAD]
        qh = q_ref[:, h * D_HEAD:(h + 1) * D_HEAD]
        s_t = jax.lax.dot_general(
            kh, qh, (((1,), (1,)), ((), ())),
            preferred_element_type=jnp.float32)
        mx = jnp.max(jnp.where(sampled, s_t, NEG), axis=0)
        sm = jnp.sum(s_t * c, axis=0)
        m_ref[h, pl.ds(qt * R, R)] = mx - sm * (1.0 / L)


def _scan_rows(x):
    incl = x
    sh = 1
    while sh < L:
        incl = incl + jnp.concatenate(
            [jnp.zeros((N_HEADS, sh), jnp.float32), incl[:, :L - sh]], axis=1)
        sh *= 2
    return incl


def _select_body(m_ref, sel_ref, pos_ref):
    m = m_ref[...]
    lo = jnp.min(m, axis=1, keepdims=True) - 1.0
    hi = jnp.max(m, axis=1, keepdims=True)
    kf = float(N_SEL)

    def step(_, carry):
        lo, hi = carry
        mid = 0.5 * (lo + hi)
        cnt = jnp.sum((m > mid).astype(jnp.float32), axis=1, keepdims=True)
        big = cnt >= kf
        return jnp.where(big, mid, lo), jnp.where(big, hi, mid)

    lo, hi = jax.lax.fori_loop(0, 60, step, (lo, hi))
    thr = jnp.max(jnp.where(m <= hi, m, NEG), axis=1, keepdims=True)
    gt = (m > thr).astype(jnp.float32)
    need = kf - jnp.sum(gt, axis=1, keepdims=True)
    tie = (m == thr).astype(jnp.float32)
    tie_excl = _scan_rows(tie) - tie
    sel = gt + tie * (tie_excl < need).astype(jnp.float32)
    sel_ref[...] = sel
    pos_ref[...] = _scan_rows(sel) - sel


_SC_MESH = plsc.VectorSubcoreMesh(core_axis_name="c", subcore_axis_name="s")


@functools.partial(
    pl.kernel, mesh=_SC_MESH,
    compiler_params=pltpu.CompilerParams(needs_layout_passes=False),
    out_type=[
        jax.ShapeDtypeStruct((N_HEADS, N_SEL, 2 * D_HEAD), jnp.float32),
        jax.ShapeDtypeStruct((N_HEADS, N_SEL), jnp.int32),
    ],
    scratch_types=[
        pltpu.VMEM((L,), jnp.float32),
        pltpu.VMEM((L,), jnp.float32),
        pltpu.VMEM((HALF,), jnp.float32),
        pltpu.VMEM((HALF,), jnp.int32),
        pltpu.VMEM((HALF,), jnp.int32),
        pltpu.VMEM((HALF, 2 * D_HEAD), jnp.float32),
        pltpu.SemaphoreType.DMA,
    ],
)
def _sc_gather(sel_hbm, pos_hbm, q2_hbm, qsel_hbm, idx_hbm,
               sel_v, pos_v, idxf_v, idx_v, gidx_v, rows_v, sem):
    wid = lax.axis_index("s") * 2 + lax.axis_index("c")
    h = wid // 2
    hf = wid % 2

    @pl.when(h < N_HEADS)
    def _():
        pltpu.sync_copy(sel_hbm.at[h], sel_v)
        pltpu.sync_copy(pos_hbm.at[h], pos_v)
        base = jnp.float32(hf * HALF)

        def body(i, _):
            s16 = sel_v[pl.ds(i * 16, 16)]
            p16 = pos_v[pl.ds(i * 16, 16)]
            i16 = lax.iota(jnp.int32, 16) + i * 16
            m = (s16 > 0.5) & (p16 >= base) & (p16 < base + HALF)
            dst = p16.astype(jnp.int32) - hf * HALF
            plsc.store_scatter(idxf_v, [dst],
                               (i16 * N_HEADS + h).astype(jnp.float32),
                               mask=m)
            return 0

        lax.fori_loop(0, L // 16, body, 0)

        def conv(i, _):
            v = idxf_v[pl.ds(i * 16, 16)].astype(jnp.int32)
            idx_v[pl.ds(i * 16, 16)] = v
            gidx_v[pl.ds(i * 16, 16)] = v // 2
            return 0

        lax.fori_loop(0, HALF // 16, conv, 0)
        for j in range(HALF // 128):
            pltpu.async_copy(
                q2_hbm.at[gidx_v.at[pl.ds(j * 128, 128)]],
                rows_v.at[pl.ds(j * 128, 128)], sem).wait()
        pltpu.sync_copy(rows_v, qsel_hbm.at[h, pl.ds(hf * HALF, HALF)])
        pltpu.sync_copy(idx_v, idx_hbm.at[h, pl.ds(hf * HALF, HALF)])


def _attn_body(qs_ref, k_ref, v_ref, out_ref):
    scale = 1.0 / math.sqrt(D_HEAD)
    for hh in range(2):
        sl = slice(hh * D_HEAD, (hh + 1) * D_HEAD)
        qs = qs_ref[hh][:, sl]
        s = jax.lax.dot_general(
            qs, k_ref[:, sl], (((1,), (1,)), ((), ())),
            preferred_element_type=jnp.float32) * scale
        mx = jnp.max(s, axis=1, keepdims=True)
        e = jnp.exp(s - mx)
        den = jnp.sum(e, axis=1, keepdims=True)
        attn = jnp.dot(e, v_ref[:, sl],
                       preferred_element_type=jnp.float32) / den
        vmean = jnp.mean(v_ref[:, sl], axis=0, keepdims=True)
        vpad = jnp.broadcast_to(vmean, (NSP - N_SEL, D_HEAD))
        out_ref[hh, 0:N_SEL, 0:D_HEAD] = attn
        out_ref[hh, 0:N_SEL, D_HEAD:2 * D_HEAD] = attn
        out_ref[hh, N_SEL:NSP, 0:D_HEAD] = vpad
        out_ref[hh, N_SEL:NSP, D_HEAD:2 * D_HEAD] = vpad


@functools.partial(
    pl.kernel, mesh=_SC_MESH,
    compiler_params=pltpu.CompilerParams(needs_layout_passes=False),
    out_type=jax.ShapeDtypeStruct((NROW, 2 * D_HEAD), jnp.float32),
    scratch_types=[
        pltpu.VMEM((N_HEADS * N_SEL,), jnp.int32),
        pltpu.VMEM((CH,), jnp.float32),
        pltpu.VMEM((CH,), jnp.int32),
        pltpu.VMEM((CH, 2 * D_HEAD), jnp.float32),
        pltpu.SemaphoreType.DMA,
    ],
)
def _sc_assemble(idxf_hbm, mapd_hbm, attnf_hbm, out_hbm, idxall_v, mapf_v,
                 map_v, chunk_v, sem):
    wid = lax.axis_index("s") * 2 + lax.axis_index("c")
    lo = wid * CH
    pltpu.sync_copy(idxf_hbm, idxall_v)
    pltpu.sync_copy(mapd_hbm.at[pl.ds(lo, CH)], mapf_v)

    for h in range(N_HEADS):
        def sbody(j, _):
            idx16 = idxall_v[pl.ds(h * N_SEL + j * 16, 16)]
            val16 = lax.iota(jnp.int32, 16) + (h * NSP + j * 16)
            m = (idx16 >= lo) & (idx16 < lo + CH)
            plsc.store_scatter(mapf_v, [idx16 - lo],
                               val16.astype(jnp.float32), mask=m)
            return 0

        lax.fori_loop(0, N_SEL // 16, sbody, 0)

    def conv(i, _):
        map_v[pl.ds(i * 16, 16)] = mapf_v[pl.ds(i * 16, 16)].astype(jnp.int32)
        return 0

    lax.fori_loop(0, CH // 16, conv, 0)

    copies = [
        pltpu.async_copy(
            attnf_hbm.at[map_v.at[pl.ds(j * 128, 128)]],
            chunk_v.at[pl.ds(j * 128, 128)], sem)
        for j in range(CH // 128)
    ]
    for c in copies:
        c.wait()
    pltpu.sync_copy(chunk_v, out_hbm.at[pl.ds(lo, CH)])


def _compact_body(w_ref, out_ref):
    for h in range(N_HEADS):
        out_ref[:, h * D_HEAD:(h + 1) * D_HEAD] = w_ref[:, h, 0:D_HEAD]


def kernel(query, key, value, Wq, bq, Wk, bk, Wv, bv):
    xq = query[0]
    xk = key[0]
    xv = value[0]
    b2 = lambda b: b.reshape(1, D_MODEL)
    counts_t = jnp.asarray(_COUNTS_T)

    q, k, v = pl.pallas_call(
        _proj_body,
        grid=(L // R,),
        in_specs=[
            pl.BlockSpec((R, D_MODEL), lambda i: (i, 0)),
            pl.BlockSpec((R, D_MODEL), lambda i: (i, 0)),
            pl.BlockSpec((R, D_MODEL), lambda i: (i, 0)),
            pl.BlockSpec((D_MODEL, D_MODEL), lambda i: (0, 0)),
            pl.BlockSpec((1, D_MODEL), lambda i: (0, 0)),
            pl.BlockSpec((D_MODEL, D_MODEL), lambda i: (0, 0)),
            pl.BlockSpec((1, D_MODEL), lambda i: (0, 0)),
            pl.BlockSpec((D_MODEL, D_MODEL), lambda i: (0, 0)),
            pl.BlockSpec((1, D_MODEL), lambda i: (0, 0)),
        ],
        out_specs=[
            pl.BlockSpec((R, D_MODEL), lambda i: (i, 0)),
            pl.BlockSpec((R, D_MODEL), lambda i: (i, 0)),
            pl.BlockSpec((R, D_MODEL), lambda i: (i, 0)),
        ],
        out_shape=[jax.ShapeDtypeStruct((L, D_MODEL), jnp.float32)] * 3,
    )(xq, xk, xv, Wq, b2(bq), Wk, b2(bk), Wv, b2(bv))

    measure = pl.pallas_call(
        _measure_body,
        grid=(L // R,),
        in_specs=[
            pl.BlockSpec((R, D_MODEL), lambda i: (i, 0)),
            pl.BlockSpec((L, D_MODEL), lambda i: (0, 0)),
            pl.BlockSpec((L, R), lambda i: (0, i)),
        ],
        out_specs=pl.BlockSpec((N_HEADS, L), lambda i: (0, 0)),
        out_shape=jax.ShapeDtypeStruct((N_HEADS, L), jnp.float32),
    )(q, k, counts_t)

    sel, pos = pl.pallas_call(
        _select_body,
        out_shape=[jax.ShapeDtypeStruct((N_HEADS, L), jnp.float32)] * 2,
    )(measure)

    q2 = q.reshape(L * N_HEADS // 2, 2 * D_HEAD)
    q_sel, idx = _sc_gather(sel, pos, q2)

    attn_ext = pl.pallas_call(
        _attn_body,
        grid=(N_HEADS // 2,),
        in_specs=[
            pl.BlockSpec((2, N_SEL, 2 * D_HEAD), lambda h: (h, 0, 0)),
            pl.BlockSpec((L, 2 * D_HEAD), lambda h: (0, h)),
            pl.BlockSpec((L, 2 * D_HEAD), lambda h: (0, h)),
        ],
        out_specs=pl.BlockSpec((2, NSP, 2 * D_HEAD), lambda h: (h, 0, 0)),
        out_shape=jax.ShapeDtypeStruct((N_HEADS, NSP, 2 * D_HEAD),
                                       jnp.float32),
    )(q_sel, k, v)

    outw = _sc_assemble(idx.reshape(-1), jnp.asarray(_MAP_DEFAULT),
                        attn_ext.reshape(N_HEADS * NSP, 2 * D_HEAD))

    out = pl.pallas_call(
        _compact_body,
        grid=(L // R,),
        in_specs=[
            pl.BlockSpec((R, N_HEADS, 2 * D_HEAD), lambda i: (i, 0, 0)),
        ],
        out_specs=pl.BlockSpec((R, D_MODEL), lambda i: (i, 0)),
        out_shape=jax.ShapeDtypeStruct((L, D_MODEL), jnp.float32),
    )(outw.reshape(L, N_HEADS, 2 * D_HEAD))

    return out[None]


# speedup vs baseline: 1.0094x; 1.0070x over previous
import functools
import math

import jax
import jax.numpy as jnp
import numpy as np
from jax import lax
from jax.experimental import pallas as pl
from jax.experimental.pallas import tpu as pltpu, tpu_sc as plsc

L = 2048
D_MODEL = 768
N_HEADS = 12
D_HEAD = 64
N_SEL = 512
R = 256
NEG = -1e30
NROW = L * N_HEADS
HALF = N_SEL // 2
NSP = N_SEL + 8
CH = NROW // 32


def _build_counts_t() -> np.ndarray:
    try:
        cpu = jax.devices("cpu")[0]
        ctx = jax.default_device(cpu)
    except Exception:
        import contextlib
        ctx = contextlib.nullcontext()
    with ctx:
        idx = np.asarray(
            jax.random.randint(jax.random.key(42), (L, N_SEL), 0, L))
    counts = np.zeros((L, L), np.float32)
    np.add.at(counts, (np.arange(L)[:, None], idx), 1.0)
    return np.ascontiguousarray(counts.T)


_COUNTS_T = _build_counts_t()
_MAP_DEFAULT = (((np.arange(L * N_HEADS) % N_HEADS) * (N_SEL + 8) + N_SEL)
                .astype(np.float32))


def _proj_body(xq_ref, xk_ref, xv_ref, wq_ref, bq_ref, wk_ref, bk_ref,
               wv_ref, bv_ref, q_ref, k_ref, v_ref):
    q_ref[...] = (
        jnp.dot(xq_ref[...], wq_ref[...], preferred_element_type=jnp.float32)
        + bq_ref[...])
    k_ref[...] = (
        jnp.dot(xk_ref[...], wk_ref[...], preferred_element_type=jnp.float32)
        + bk_ref[...])
    v_ref[...] = (
        jnp.dot(xv_ref[...], wv_ref[...], preferred_element_type=jnp.float32)
        + bv_ref[...])


def _measure_body(q_ref, k_ref, ct_ref, m_ref):
    qt = pl.program_id(0)
    c = ct_ref[...]
    sampled = c > 0.0
    for h in range(N_HEADS):
        kh = k_ref[:, h * D_HEAD:(h + 1) * D_HEAD]
        qh = q_ref[:, h * D_HEAD:(h + 1) * D_HEAD]
        s_t = jax.lax.dot_general(
            kh, qh, (((1,), (1,)), ((), ())),
            preferred_element_type=jnp.float32)
        mx = jnp.max(jnp.where(sampled, s_t, NEG), axis=0)
        sm = jnp.sum(s_t * c, axis=0)
        m_ref[h, pl.ds(qt * R, R)] = mx - sm * (1.0 / L)


def _scan_rows(x):
    incl = x
    sh = 1
    while sh < L:
        incl = incl + jnp.concatenate(
            [jnp.zeros((N_HEADS, sh), jnp.float32), incl[:, :L - sh]], axis=1)
        sh *= 2
    return incl


def _select_body(m_ref, sel_ref, pos_ref, mapd_ref):
    m = m_ref[...]
    lo = jnp.min(m, axis=1, keepdims=True) - 1.0
    hi = jnp.max(m, axis=1, keepdims=True)
    kf = float(N_SEL)

    def step(_, carry):
        lo, hi = carry
        mid = 0.5 * (lo + hi)
        cnt = jnp.sum((m > mid).astype(jnp.float32), axis=1, keepdims=True)
        big = cnt >= kf
        return jnp.where(big, mid, lo), jnp.where(big, hi, mid)

    lo, hi = jax.lax.fori_loop(0, 60, step, (lo, hi))
    thr = jnp.max(jnp.where(m <= hi, m, NEG), axis=1, keepdims=True)
    gt = (m > thr).astype(jnp.float32)
    need = kf - jnp.sum(gt, axis=1, keepdims=True)
    tie = (m == thr).astype(jnp.float32)
    tie_excl = _scan_rows(tie) - tie
    sel = gt + tie * (tie_excl < need).astype(jnp.float32)
    sel_ref[...] = sel
    pos = _scan_rows(sel) - sel
    pos_ref[...] = pos
    hid = jax.lax.broadcasted_iota(jnp.int32, (N_HEADS, L), 0).astype(
        jnp.float32)
    map2d = hid * float(NSP) + jnp.where(sel > 0.5, pos, float(N_SEL))
    mapd_ref[...] = jnp.transpose(map2d)


_SC_MESH = plsc.VectorSubcoreMesh(core_axis_name="c", subcore_axis_name="s")


@functools.partial(
    pl.kernel, mesh=_SC_MESH,
    compiler_params=pltpu.CompilerParams(needs_layout_passes=False),
    out_type=jax.ShapeDtypeStruct((N_HEADS, N_SEL, 2 * D_HEAD),
                                  jnp.float32),
    scratch_types=[
        pltpu.VMEM((L,), jnp.float32),
        pltpu.VMEM((L,), jnp.float32),
        pltpu.VMEM((HALF,), jnp.float32),
        pltpu.VMEM((HALF,), jnp.int32),
        pltpu.VMEM((HALF,), jnp.int32),
        pltpu.VMEM((HALF, 2 * D_HEAD), jnp.float32),
        pltpu.SemaphoreType.DMA,
    ],
)
def _sc_gather(sel_hbm, pos_hbm, q2_hbm, qsel_hbm,
               sel_v, pos_v, idxf_v, idx_v, gidx_v, rows_v, sem):
    wid = lax.axis_index("s") * 2 + lax.axis_index("c")
    h = wid // 2
    hf = wid % 2

    @pl.when(h < N_HEADS)
    def _():
        pltpu.sync_copy(sel_hbm.at[h], sel_v)
        pltpu.sync_copy(pos_hbm.at[h], pos_v)
        base = jnp.float32(hf * HALF)

        def body(i, _):
            s16 = sel_v[pl.ds(i * 16, 16)]
            p16 = pos_v[pl.ds(i * 16, 16)]
            i16 = lax.iota(jnp.int32, 16) + i * 16
            m = (s16 > 0.5) & (p16 >= base) & (p16 < base + HALF)
            dst = p16.astype(jnp.int32) - hf * HALF
            plsc.store_scatter(idxf_v, [dst],
                               (i16 * N_HEADS + h).astype(jnp.float32),
                               mask=m)
            return 0

        lax.fori_loop(0, L // 16, body, 0)

        def conv(i, _):
            v = idxf_v[pl.ds(i * 16, 16)].astype(jnp.int32)
            idx_v[pl.ds(i * 16, 16)] = v
            gidx_v[pl.ds(i * 16, 16)] = v // 2
            return 0

        lax.fori_loop(0, HALF // 16, conv, 0)
        for j in range(HALF // 128):
            pltpu.async_copy(
                q2_hbm.at[gidx_v.at[pl.ds(j * 128, 128)]],
                rows_v.at[pl.ds(j * 128, 128)], sem).wait()
        pltpu.sync_copy(rows_v, qsel_hbm.at[h, pl.ds(hf * HALF, HALF)])


def _attn_body(qs_ref, k_ref, v_ref, out_ref):
    scale = 1.0 / math.sqrt(D_HEAD)
    for hh in range(2):
        sl = slice(hh * D_HEAD, (hh + 1) * D_HEAD)
        qs = qs_ref[hh][:, sl]
        s = jax.lax.dot_general(
            qs, k_ref[:, sl], (((1,), (1,)), ((), ())),
            preferred_element_type=jnp.float32) * scale
        mx = jnp.max(s, axis=1, keepdims=True)
        e = jnp.exp(s - mx)
        den = jnp.sum(e, axis=1, keepdims=True)
        attn = jnp.dot(e, v_ref[:, sl],
                       preferred_element_type=jnp.float32) / den
        vmean = jnp.mean(v_ref[:, sl], axis=0, keepdims=True)
        vpad = jnp.broadcast_to(vmean, (NSP - N_SEL, D_HEAD))
        out_ref[hh, 0:N_SEL, 0:D_HEAD] = attn
        out_ref[hh, 0:N_SEL, D_HEAD:2 * D_HEAD] = attn
        out_ref[hh, N_SEL:NSP, 0:D_HEAD] = vpad
        out_ref[hh, N_SEL:NSP, D_HEAD:2 * D_HEAD] = vpad


@functools.partial(
    pl.kernel, mesh=_SC_MESH,
    compiler_params=pltpu.CompilerParams(needs_layout_passes=False),
    out_type=jax.ShapeDtypeStruct((NROW, 2 * D_HEAD), jnp.float32),
    scratch_types=[
        pltpu.VMEM((CH,), jnp.float32),
        pltpu.VMEM((CH,), jnp.int32),
        pltpu.VMEM((CH, 2 * D_HEAD), jnp.float32),
        pltpu.SemaphoreType.DMA,
    ],
)
def _sc_assemble(mapd_hbm, attnf_hbm, out_hbm, mapf_v, map_v, chunk_v, sem):
    wid = lax.axis_index("s") * 2 + lax.axis_index("c")
    lo = wid * CH
    pltpu.sync_copy(mapd_hbm.at[pl.ds(lo, CH)], mapf_v)

    def conv(i, _):
        map_v[pl.ds(i * 16, 16)] = mapf_v[pl.ds(i * 16, 16)].astype(jnp.int32)
        return 0

    lax.fori_loop(0, CH // 16, conv, 0)

    copies = [
        pltpu.async_copy(
            attnf_hbm.at[map_v.at[pl.ds(j * 128, 128)]],
            chunk_v.at[pl.ds(j * 128, 128)], sem)
        for j in range(CH // 128)
    ]
    for c in copies:
        c.wait()
    pltpu.sync_copy(chunk_v, out_hbm.at[pl.ds(lo, CH)])


def _compact_body(w_ref, out_ref):
    for h in range(N_HEADS):
        out_ref[:, h * D_HEAD:(h + 1) * D_HEAD] = w_ref[:, h, 0:D_HEAD]


def kernel(query, key, value, Wq, bq, Wk, bk, Wv, bv):
    xq = query[0]
    xk = key[0]
    xv = value[0]
    b2 = lambda b: b.reshape(1, D_MODEL)
    counts_t = jnp.asarray(_COUNTS_T)

    q, k, v = pl.pallas_call(
        _proj_body,
        grid=(L // R,),
        in_specs=[
            pl.BlockSpec((R, D_MODEL), lambda i: (i, 0)),
            pl.BlockSpec((R, D_MODEL), lambda i: (i, 0)),
            pl.BlockSpec((R, D_MODEL), lambda i: (i, 0)),
            pl.BlockSpec((D_MODEL, D_MODEL), lambda i: (0, 0)),
            pl.BlockSpec((1, D_MODEL), lambda i: (0, 0)),
            pl.BlockSpec((D_MODEL, D_MODEL), lambda i: (0, 0)),
            pl.BlockSpec((1, D_MODEL), lambda i: (0, 0)),
            pl.BlockSpec((D_MODEL, D_MODEL), lambda i: (0, 0)),
            pl.BlockSpec((1, D_MODEL), lambda i: (0, 0)),
        ],
        out_specs=[
            pl.BlockSpec((R, D_MODEL), lambda i: (i, 0)),
            pl.BlockSpec((R, D_MODEL), lambda i: (i, 0)),
            pl.BlockSpec((R, D_MODEL), lambda i: (i, 0)),
        ],
        out_shape=[jax.ShapeDtypeStruct((L, D_MODEL), jnp.float32)] * 3,
    )(xq, xk, xv, Wq, b2(bq), Wk, b2(bk), Wv, b2(bv))

    measure = pl.pallas_call(
        _measure_body,
        grid=(L // R,),
        in_specs=[
            pl.BlockSpec((R, D_MODEL), lambda i: (i, 0)),
            pl.BlockSpec((L, D_MODEL), lambda i: (0, 0)),
            pl.BlockSpec((L, R), lambda i: (0, i)),
        ],
        out_specs=pl.BlockSpec((N_HEADS, L), lambda i: (0, 0)),
        out_shape=jax.ShapeDtypeStruct((N_HEADS, L), jnp.float32),
    )(q, k, counts_t)

    sel, pos, mapd = pl.pallas_call(
        _select_body,
        out_shape=[
            jax.ShapeDtypeStruct((N_HEADS, L), jnp.float32),
            jax.ShapeDtypeStruct((N_HEADS, L), jnp.float32),
            jax.ShapeDtypeStruct((L, N_HEADS), jnp.float32),
        ],
    )(measure)

    q2 = q.reshape(L * N_HEADS // 2, 2 * D_HEAD)
    q_sel = _sc_gather(sel, pos, q2)

    attn_ext = pl.pallas_call(
        _attn_body,
        grid=(N_HEADS // 2,),
        in_specs=[
            pl.BlockSpec((2, N_SEL, 2 * D_HEAD), lambda h: (h, 0, 0)),
            pl.BlockSpec((L, 2 * D_HEAD), lambda h: (0, h)),
            pl.BlockSpec((L, 2 * D_HEAD), lambda h: (0, h)),
        ],
        out_specs=pl.BlockSpec((2, NSP, 2 * D_HEAD), lambda h: (h, 0, 0)),
        out_shape=jax.ShapeDtypeStruct((N_HEADS, NSP, 2 * D_HEAD),
                                       jnp.float32),
    )(q_sel, k, v)

    outw = _sc_assemble(mapd.reshape(-1),
                        attn_ext.reshape(N_HEADS * NSP, 2 * D_HEAD))

    out = pl.pallas_call(
        _compact_body,
        grid=(L // R,),
        in_specs=[
            pl.BlockSpec((R, N_HEADS, 2 * D_HEAD), lambda i: (i, 0, 0)),
        ],
        out_specs=pl.BlockSpec((R, D_MODEL), lambda i: (i, 0)),
        out_shape=jax.ShapeDtypeStruct((L, D_MODEL), jnp.float32),
    )(outw.reshape(L, N_HEADS, 2 * D_HEAD))

    return out[None]


# speedup vs baseline: 1.5612x; 1.5466x over previous
import functools
import math

import jax
import jax.numpy as jnp
import numpy as np
from jax import lax
from jax.experimental import pallas as pl
from jax.experimental.pallas import tpu as pltpu, tpu_sc as plsc

L = 2048
D_MODEL = 768
N_HEADS = 12
D_HEAD = 64
N_SEL = 512
R = 256
NEG = -1e30
NROW = L * N_HEADS
HALF = N_SEL // 2
NSP = N_SEL + 8
CH = NROW // 32


def _build_counts_t() -> np.ndarray:
    try:
        cpu = jax.devices("cpu")[0]
        ctx = jax.default_device(cpu)
    except Exception:
        import contextlib
        ctx = contextlib.nullcontext()
    with ctx:
        idx = np.asarray(
            jax.random.randint(jax.random.key(42), (L, N_SEL), 0, L))
    counts = np.zeros((L, L), np.float32)
    np.add.at(counts, (np.arange(L)[:, None], idx), 1.0)
    return np.ascontiguousarray(counts.T)


_COUNTS_T = _build_counts_t()
_MAP_DEFAULT = (((np.arange(L * N_HEADS) % N_HEADS) * (N_SEL + 8) + N_SEL)
                .astype(np.float32))


def _proj_body(xq_ref, xk_ref, xv_ref, wq_ref, bq_ref, wk_ref, bk_ref,
               wv_ref, bv_ref, q_ref, k_ref, v_ref):
    q_ref[...] = (
        jnp.dot(xq_ref[...], wq_ref[...], preferred_element_type=jnp.float32)
        + bq_ref[...])
    k_ref[...] = (
        jnp.dot(xk_ref[...], wk_ref[...], preferred_element_type=jnp.float32)
        + bk_ref[...])
    v_ref[...] = (
        jnp.dot(xv_ref[...], wv_ref[...], preferred_element_type=jnp.float32)
        + bv_ref[...])


def _measure_body(q_ref, k_ref, ct_ref, m_ref):
    qt = pl.program_id(0)
    c = ct_ref[...]
    sampled = c > 0.0
    for h in range(N_HEADS):
        kh = k_ref[:, h * D_HEAD:(h + 1) * D_HEAD]
        qh = q_ref[:, h * D_HEAD:(h + 1) * D_HEAD]
        s_t = jax.lax.dot_general(
            kh, qh, (((1,), (1,)), ((), ())),
            preferred_element_type=jnp.float32)
        mx = jnp.max(jnp.where(sampled, s_t, NEG), axis=0)
        sm = jnp.sum(s_t * c, axis=0)
        m_ref[h, pl.ds(qt * R, R)] = mx - sm * (1.0 / L)


def _scan_rows(x):
    incl = x
    sh = 1
    while sh < L:
        incl = incl + jnp.concatenate(
            [jnp.zeros((N_HEADS, sh), jnp.float32), incl[:, :L - sh]], axis=1)
        sh *= 2
    return incl


def _select_body(m_ref, sel_ref, pos_ref, selb_ref, posb_ref):
    m = m_ref[...]
    lo = jnp.min(m, axis=1, keepdims=True) - 1.0
    hi = jnp.max(m, axis=1, keepdims=True)
    kf = float(N_SEL)

    def step(_, carry):
        lo, hi = carry
        mid = 0.5 * (lo + hi)
        cnt = jnp.sum((m > mid).astype(jnp.float32), axis=1, keepdims=True)
        big = cnt >= kf
        return jnp.where(big, mid, lo), jnp.where(big, hi, mid)

    lo, hi = jax.lax.fori_loop(0, 60, step, (lo, hi))
    thr = jnp.max(jnp.where(m <= hi, m, NEG), axis=1, keepdims=True)
    gt = (m > thr).astype(jnp.float32)
    need = kf - jnp.sum(gt, axis=1, keepdims=True)
    tie = (m == thr).astype(jnp.float32)
    tie_excl = _scan_rows(tie) - tie
    sel = gt + tie * (tie_excl < need).astype(jnp.float32)
    sel_ref[...] = sel
    pos = _scan_rows(sel) - sel
    pos_ref[...] = pos
    col = jax.lax.broadcasted_iota(jnp.int32, (D_MODEL, N_HEADS), 0)
    hid = jax.lax.broadcasted_iota(jnp.int32, (D_MODEL, N_HEADS), 1)
    expand = (col // D_HEAD == hid).astype(jnp.float32)
    dn = (((0,), (1,)), ((), ()))
    selb_ref[...] = jax.lax.dot_general(
        sel, expand, dn, precision=jax.lax.Precision.HIGHEST,
        preferred_element_type=jnp.float32)
    posb_ref[...] = jax.lax.dot_general(
        pos, expand, dn, precision=jax.lax.Precision.HIGHEST,
        preferred_element_type=jnp.float32)


_SC_MESH = plsc.VectorSubcoreMesh(core_axis_name="c", subcore_axis_name="s")


@functools.partial(
    pl.kernel, mesh=_SC_MESH,
    compiler_params=pltpu.CompilerParams(needs_layout_passes=False),
    out_type=jax.ShapeDtypeStruct((N_HEADS, N_SEL, 2 * D_HEAD),
                                  jnp.float32),
    scratch_types=[
        pltpu.VMEM((L,), jnp.float32),
        pltpu.VMEM((L,), jnp.float32),
        pltpu.VMEM((HALF,), jnp.float32),
        pltpu.VMEM((HALF,), jnp.int32),
        pltpu.VMEM((HALF,), jnp.int32),
        pltpu.VMEM((HALF, 2 * D_HEAD), jnp.float32),
        pltpu.SemaphoreType.DMA,
    ],
)
def _sc_gather(sel_hbm, pos_hbm, q2_hbm, qsel_hbm,
               sel_v, pos_v, idxf_v, idx_v, gidx_v, rows_v, sem):
    wid = lax.axis_index("s") * 2 + lax.axis_index("c")
    h = wid // 2
    hf = wid % 2

    @pl.when(h < N_HEADS)
    def _():
        pltpu.sync_copy(sel_hbm.at[h], sel_v)
        pltpu.sync_copy(pos_hbm.at[h], pos_v)
        base = jnp.float32(hf * HALF)

        def body(i, _):
            s16 = sel_v[pl.ds(i * 16, 16)]
            p16 = pos_v[pl.ds(i * 16, 16)]
            i16 = lax.iota(jnp.int32, 16) + i * 16
            m = (s16 > 0.5) & (p16 >= base) & (p16 < base + HALF)
            dst = p16.astype(jnp.int32) - hf * HALF
            plsc.store_scatter(idxf_v, [dst],
                               (i16 * N_HEADS + h).astype(jnp.float32),
                               mask=m)
            return 0

        lax.fori_loop(0, L // 16, body, 0)

        def conv(i, _):
            v = idxf_v[pl.ds(i * 16, 16)].astype(jnp.int32)
            idx_v[pl.ds(i * 16, 16)] = v
            gidx_v[pl.ds(i * 16, 16)] = v // 2
            return 0

        lax.fori_loop(0, HALF // 16, conv, 0)
        for j in range(HALF // 128):
            pltpu.async_copy(
                q2_hbm.at[gidx_v.at[pl.ds(j * 128, 128)]],
                rows_v.at[pl.ds(j * 128, 128)], sem).wait()
        pltpu.sync_copy(rows_v, qsel_hbm.at[h, pl.ds(hf * HALF, HALF)])


def _attn_body(qs_ref, k_ref, v_ref, selb_ref, posb_ref, out_ref):
    scale = 1.0 / math.sqrt(D_HEAD)
    rid = jax.lax.broadcasted_iota(jnp.int32, (1, N_SEL), 1)
    for hh in range(2):
        sl = slice(hh * D_HEAD, (hh + 1) * D_HEAD)
        qs = qs_ref[hh][:, sl]
        s = jax.lax.dot_general(
            qs, k_ref[:, sl], (((1,), (1,)), ((), ())),
            preferred_element_type=jnp.float32) * scale
        mx = jnp.max(s, axis=1, keepdims=True)
        e = jnp.exp(s - mx)
        den = jnp.sum(e, axis=1, keepdims=True)
        attn = jnp.dot(e, v_ref[:, sl],
                       preferred_element_type=jnp.float32) / den
        selc = selb_ref[:, hh * D_HEAD:hh * D_HEAD + 1]
        posc = posb_ref[:, hh * D_HEAD:hh * D_HEAD + 1]
        g = (posc.astype(jnp.int32) == rid).astype(jnp.float32) * selc
        scat = jnp.dot(g, attn, preferred_element_type=jnp.float32)
        vmean = jnp.mean(v_ref[:, sl], axis=0, keepdims=True)
        out_ref[:, sl] = scat + vmean * (1.0 - selc)


def kernel(query, key, value, Wq, bq, Wk, bk, Wv, bv):
    xq = query[0]
    xk = key[0]
    xv = value[0]
    b2 = lambda b: b.reshape(1, D_MODEL)
    counts_t = jnp.asarray(_COUNTS_T)

    q, k, v = pl.pallas_call(
        _proj_body,
        grid=(L // R,),
        in_specs=[
            pl.BlockSpec((R, D_MODEL), lambda i: (i, 0)),
            pl.BlockSpec((R, D_MODEL), lambda i: (i, 0)),
            pl.BlockSpec((R, D_MODEL), lambda i: (i, 0)),
            pl.BlockSpec((D_MODEL, D_MODEL), lambda i: (0, 0)),
            pl.BlockSpec((1, D_MODEL), lambda i: (0, 0)),
            pl.BlockSpec((D_MODEL, D_MODEL), lambda i: (0, 0)),
            pl.BlockSpec((1, D_MODEL), lambda i: (0, 0)),
            pl.BlockSpec((D_MODEL, D_MODEL), lambda i: (0, 0)),
            pl.BlockSpec((1, D_MODEL), lambda i: (0, 0)),
        ],
        out_specs=[
            pl.BlockSpec((R, D_MODEL), lambda i: (i, 0)),
            pl.BlockSpec((R, D_MODEL), lambda i: (i, 0)),
            pl.BlockSpec((R, D_MODEL), lambda i: (i, 0)),
        ],
        out_shape=[jax.ShapeDtypeStruct((L, D_MODEL), jnp.float32)] * 3,
    )(xq, xk, xv, Wq, b2(bq), Wk, b2(bk), Wv, b2(bv))

    measure = pl.pallas_call(
        _measure_body,
        grid=(L // R,),
        in_specs=[
            pl.BlockSpec((R, D_MODEL), lambda i: (i, 0)),
            pl.BlockSpec((L, D_MODEL), lambda i: (0, 0)),
            pl.BlockSpec((L, R), lambda i: (0, i)),
        ],
        out_specs=pl.BlockSpec((N_HEADS, L), lambda i: (0, 0)),
        out_shape=jax.ShapeDtypeStruct((N_HEADS, L), jnp.float32),
    )(q, k, counts_t)

    sel, pos, selb, posb = pl.pallas_call(
        _select_body,
        out_shape=[
            jax.ShapeDtypeStruct((N_HEADS, L), jnp.float32),
            jax.ShapeDtypeStruct((N_HEADS, L), jnp.float32),
            jax.ShapeDtypeStruct((L, D_MODEL), jnp.float32),
            jax.ShapeDtypeStruct((L, D_MODEL), jnp.float32),
        ],
    )(measure)

    q2 = q.reshape(L * N_HEADS // 2, 2 * D_HEAD)
    q_sel = _sc_gather(sel, pos, q2)

    out = pl.pallas_call(
        _attn_body,
        grid=(N_HEADS // 2,),
        in_specs=[
            pl.BlockSpec((2, N_SEL, 2 * D_HEAD), lambda h: (h, 0, 0)),
            pl.BlockSpec((L, 2 * D_HEAD), lambda h: (0, h)),
            pl.BlockSpec((L, 2 * D_HEAD), lambda h: (0, h)),
            pl.BlockSpec((L, 2 * D_HEAD), lambda h: (0, h)),
            pl.BlockSpec((L, 2 * D_HEAD), lambda h: (0, h)),
        ],
        out_specs=pl.BlockSpec((L, 2 * D_HEAD), lambda h: (0, h)),
        out_shape=jax.ShapeDtypeStruct((L, D_MODEL), jnp.float32),
    )(q_sel, k, v, selb, posb)

    return out[None]


# speedup vs baseline: 1.5623x; 1.0007x over previous
import functools
import math

import jax
import jax.numpy as jnp
import numpy as np
from jax import lax
from jax.experimental import pallas as pl
from jax.experimental.pallas import tpu as pltpu, tpu_sc as plsc

L = 2048
D_MODEL = 768
N_HEADS = 12
D_HEAD = 64
N_SEL = 512
R = 256
NEG = -1e30
HALF = N_SEL // 2


def _build_counts_t() -> np.ndarray:
    try:
        cpu = jax.devices("cpu")[0]
        ctx = jax.default_device(cpu)
    except Exception:
        import contextlib
        ctx = contextlib.nullcontext()
    with ctx:
        idx = np.asarray(
            jax.random.randint(jax.random.key(42), (L, N_SEL), 0, L))
    counts = np.zeros((L, L), np.float32)
    np.add.at(counts, (np.arange(L)[:, None], idx), 1.0)
    return np.ascontiguousarray(counts.T)


_COUNTS_T = _build_counts_t()


def _proj_body(xq_ref, xk_ref, xv_ref, wq_ref, bq_ref, wk_ref, bk_ref,
               wv_ref, bv_ref, q_ref, k_ref, v_ref):
    q_ref[...] = (
        jnp.dot(xq_ref[...], wq_ref[...], preferred_element_type=jnp.float32)
        + bq_ref[...])
    k_ref[...] = (
        jnp.dot(xk_ref[...], wk_ref[...], preferred_element_type=jnp.float32)
        + bk_ref[...])
    v_ref[...] = (
        jnp.dot(xv_ref[...], wv_ref[...], preferred_element_type=jnp.float32)
        + bv_ref[...])


def _measure_body(q_ref, k_ref, ct_ref, m_ref):
    qt = pl.program_id(0)
    c = ct_ref[...]
    sampled = c > 0.0
    for h in range(N_HEADS):
        kh = k_ref[:, h * D_HEAD:(h + 1) * D_HEAD]
        qh = q_ref[:, h * D_HEAD:(h + 1) * D_HEAD]
        s_t = jax.lax.dot_general(
            kh, qh, (((1,), (1,)), ((), ())),
            preferred_element_type=jnp.float32)
        mx = jnp.max(jnp.where(sampled, s_t, NEG), axis=0)
        sm = jnp.sum(s_t * c, axis=0)
        m_ref[h, pl.ds(qt * R, R)] = mx - sm * (1.0 / L)


def _scan_rows(x):
    incl = x
    sh = 1
    while sh < L:
        incl = incl + jnp.concatenate(
            [jnp.zeros((N_HEADS, sh), jnp.float32), incl[:, :L - sh]], axis=1)
        sh *= 2
    return incl


def _select_body(m_ref, sel_ref, pos_ref, selb_ref, posb_ref):
    m = m_ref[...]
    lo = jnp.min(m, axis=1, keepdims=True) - 1.0
    hi = jnp.max(m, axis=1, keepdims=True)
    kf = float(N_SEL)

    def step(_, carry):
        lo, hi = carry
        mid = 0.5 * (lo + hi)
        cnt = jnp.sum((m > mid).astype(jnp.float32), axis=1, keepdims=True)
        big = cnt >= kf
        return jnp.where(big, mid, lo), jnp.where(big, hi, mid)

    lo, hi = jax.lax.fori_loop(0, 60, step, (lo, hi))
    thr = jnp.max(jnp.where(m <= hi, m, NEG), axis=1, keepdims=True)
    gt = (m > thr).astype(jnp.float32)
    need = kf - jnp.sum(gt, axis=1, keepdims=True)
    tie = (m == thr).astype(jnp.float32)
    tie_excl = _scan_rows(tie) - tie
    sel = gt + tie * (tie_excl < need).astype(jnp.float32)
    sel_ref[...] = sel
    pos = _scan_rows(sel) - sel
    pos_ref[...] = pos
    col = jax.lax.broadcasted_iota(jnp.int32, (D_MODEL, N_HEADS), 0)
    hid = jax.lax.broadcasted_iota(jnp.int32, (D_MODEL, N_HEADS), 1)
    expand = (col // D_HEAD == hid).astype(jnp.float32)
    dn = (((0,), (1,)), ((), ()))
    selb_ref[...] = jax.lax.dot_general(
        sel, expand, dn, precision=jax.lax.Precision.HIGHEST,
        preferred_element_type=jnp.float32)
    posb_ref[...] = jax.lax.dot_general(
        pos, expand, dn, precision=jax.lax.Precision.HIGHEST,
        preferred_element_type=jnp.float32)


_SC_MESH = plsc.VectorSubcoreMesh(core_axis_name="c", subcore_axis_name="s")


@functools.partial(
    pl.kernel, mesh=_SC_MESH,
    compiler_params=pltpu.CompilerParams(needs_layout_passes=False),
    out_type=jax.ShapeDtypeStruct((N_HEADS, N_SEL, 2 * D_HEAD),
                                  jnp.float32),
    scratch_types=[
        pltpu.VMEM((L,), jnp.float32),
        pltpu.VMEM((L,), jnp.float32),
        pltpu.VMEM((HALF,), jnp.float32),
        pltpu.VMEM((HALF,), jnp.int32),
        pltpu.VMEM((HALF,), jnp.int32),
        pltpu.VMEM((HALF, 2 * D_HEAD), jnp.float32),
        pltpu.SemaphoreType.DMA,
    ],
)
def _sc_gather(sel_hbm, pos_hbm, q2_hbm, qsel_hbm,
               sel_v, pos_v, idxf_v, idx_v, gidx_v, rows_v, sem):
    wid = lax.axis_index("s") * 2 + lax.axis_index("c")
    h = wid // 2
    hf = wid % 2

    @pl.when(h < N_HEADS)
    def _():
        pltpu.sync_copy(sel_hbm.at[h], sel_v)
        pltpu.sync_copy(pos_hbm.at[h], pos_v)
        base = jnp.float32(hf * HALF)

        def body(i, _):
            s16 = sel_v[pl.ds(i * 16, 16)]
            p16 = pos_v[pl.ds(i * 16, 16)]
            i16 = lax.iota(jnp.int32, 16) + i * 16
            m = (s16 > 0.5) & (p16 >= base) & (p16 < base + HALF)
            dst = p16.astype(jnp.int32) - hf * HALF
            plsc.store_scatter(idxf_v, [dst],
                               (i16 * N_HEADS + h).astype(jnp.float32),
                               mask=m)
            return 0

        lax.fori_loop(0, L // 16, body, 0)

        def conv(i, _):
            v = idxf_v[pl.ds(i * 16, 16)].astype(jnp.int32)
            idx_v[pl.ds(i * 16, 16)] = v
            gidx_v[pl.ds(i * 16, 16)] = v // 2
            return 0

        lax.fori_loop(0, HALF // 16, conv, 0)
        for j in range(HALF // 128):
            pltpu.async_copy(
                q2_hbm.at[gidx_v.at[pl.ds(j * 128, 128)]],
                rows_v.at[pl.ds(j * 128, 128)], sem).wait()
        pltpu.sync_copy(rows_v, qsel_hbm.at[h, pl.ds(hf * HALF, HALF)])


def _attn_body(qs_ref, k_ref, v_ref, selb_ref, posb_ref, out_ref):
    scale = 1.0 / math.sqrt(D_HEAD)
    rid = jax.lax.broadcasted_iota(jnp.int32, (1, N_SEL), 1)
    for hh in range(2):
        sl = slice(hh * D_HEAD, (hh + 1) * D_HEAD)
        qs = qs_ref[hh][:, sl]
        s = jax.lax.dot_general(
            qs, k_ref[:, sl], (((1,), (1,)), ((), ())),
            preferred_element_type=jnp.float32) * scale
        mx = jnp.max(s, axis=1, keepdims=True)
        e = jnp.exp(s - mx)
        den = jnp.sum(e, axis=1, keepdims=True)
        attn = jnp.dot(e, v_ref[:, sl],
                       preferred_element_type=jnp.float32) / den
        selc = selb_ref[:, hh * D_HEAD:hh * D_HEAD + 1]
        posc = posb_ref[:, hh * D_HEAD:hh * D_HEAD + 1]
        g = (posc.astype(jnp.int32) == rid).astype(jnp.float32) * selc
        scat = jnp.dot(g, attn, preferred_element_type=jnp.float32)
        vmean = jnp.mean(v_ref[:, sl], axis=0, keepdims=True)
        out_ref[:, sl] = scat + vmean * (1.0 - selc)


def kernel(query, key, value, Wq, bq, Wk, bk, Wv, bv):
    xq = query[0]
    xk = key[0]
    xv = value[0]
    b2 = lambda b: b.reshape(1, D_MODEL)
    counts_t = jnp.asarray(_COUNTS_T)

    q, k, v = pl.pallas_call(
        _proj_body,
        grid=(L // R,),
        in_specs=[
            pl.BlockSpec((R, D_MODEL), lambda i: (i, 0)),
            pl.BlockSpec((R, D_MODEL), lambda i: (i, 0)),
            pl.BlockSpec((R, D_MODEL), lambda i: (i, 0)),
            pl.BlockSpec((D_MODEL, D_MODEL), lambda i: (0, 0)),
            pl.BlockSpec((1, D_MODEL), lambda i: (0, 0)),
            pl.BlockSpec((D_MODEL, D_MODEL), lambda i: (0, 0)),
            pl.BlockSpec((1, D_MODEL), lambda i: (0, 0)),
            pl.BlockSpec((D_MODEL, D_MODEL), lambda i: (0, 0)),
            pl.BlockSpec((1, D_MODEL), lambda i: (0, 0)),
        ],
        out_specs=[
            pl.BlockSpec((R, D_MODEL), lambda i: (i, 0)),
            pl.BlockSpec((R, D_MODEL), lambda i: (i, 0)),
            pl.BlockSpec((R, D_MODEL), lambda i: (i, 0)),
        ],
        out_shape=[jax.ShapeDtypeStruct((L, D_MODEL), jnp.float32)] * 3,
    )(xq, xk, xv, Wq, b2(bq), Wk, b2(bk), Wv, b2(bv))

    measure = pl.pallas_call(
        _measure_body,
        grid=(L // R,),
        in_specs=[
            pl.BlockSpec((R, D_MODEL), lambda i: (i, 0)),
            pl.BlockSpec((L, D_MODEL), lambda i: (0, 0)),
            pl.BlockSpec((L, R), lambda i: (0, i)),
        ],
        out_specs=pl.BlockSpec((N_HEADS, L), lambda i: (0, 0)),
        out_shape=jax.ShapeDtypeStruct((N_HEADS, L), jnp.float32),
    )(q, k, counts_t)

    sel, pos, selb, posb = pl.pallas_call(
        _select_body,
        out_shape=[
            jax.ShapeDtypeStruct((N_HEADS, L), jnp.float32),
            jax.ShapeDtypeStruct((N_HEADS, L), jnp.float32),
            jax.ShapeDtypeStruct((L, D_MODEL), jnp.float32),
            jax.ShapeDtypeStruct((L, D_MODEL), jnp.float32),
        ],
    )(measure)

    q2 = q.reshape(L * N_HEADS // 2, 2 * D_HEAD)
    q_sel = _sc_gather(sel, pos, q2)

    out = pl.pallas_call(
        _attn_body,
        grid=(N_HEADS // 2,),
        in_specs=[
            pl.BlockSpec((2, N_SEL, 2 * D_HEAD), lambda h: (h, 0, 0)),
            pl.BlockSpec((L, 2 * D_HEAD), lambda h: (0, h)),
            pl.BlockSpec((L, 2 * D_HEAD), lambda h: (0, h)),
            pl.BlockSpec((L, 2 * D_HEAD), lambda h: (0, h)),
            pl.BlockSpec((L, 2 * D_HEAD), lambda h: (0, h)),
        ],
        out_specs=pl.BlockSpec((L, 2 * D_HEAD), lambda h: (0, h)),
        out_shape=jax.ShapeDtypeStruct((L, D_MODEL), jnp.float32),
    )(q_sel, k, v, selb, posb)

    return out[None]
